# baseline (speedup 1.0000x reference)
import jax
import jax.numpy as jnp
from jax import lax
from jax.experimental import pallas as pl
from jax.experimental.pallas import tpu as pltpu
from jax.experimental.pallas import tpu_sc as plsc

BATCH = 16384
DIM = 16
VOCAB = 1000000

_NC = 2
_NS = 16
_NW = _NC * _NS
_ROWS_PER_W = BATCH // _NW
_EL_PER_W = _ROWS_PER_W * DIM
_CHUNK = 128
_NCHUNK = _EL_PER_W // _CHUNK
_NGROUP = _ROWS_PER_W // 16
_WAVE = 16


def _tt_body(x_hbm, y_hbm, art_hbm, cust_hbm, out_hbm,
             xidx, yidx, xe, ye, xrows, yrows, out_v, sem):
    wid = lax.axis_index("s") * _NC + lax.axis_index("c")
    base = wid * _ROWS_PER_W

    cp_x = pltpu.make_async_copy(x_hbm.at[pl.ds(base, _ROWS_PER_W)], xidx, sem)
    cp_y = pltpu.make_async_copy(y_hbm.at[pl.ds(base, _ROWS_PER_W)], yidx, sem)
    cp_x.start()
    cp_y.start()
    cp_x.wait()
    cp_y.wait()

    iota = lax.iota(jnp.int32, 16)
    stride = iota * VOCAB

    def expand(g, carry):
        vx = xidx[pl.ds(g * 16, 16)]
        vy = yidx[pl.ds(g * 16, 16)]
        for k in range(16):
            b16 = (g * 16 + k) * 16
            xe[pl.ds(b16, 16)] = jnp.full((16,), vx[k], jnp.int32) + stride
            ye[pl.ds(b16, 16)] = jnp.full((16,), vy[k], jnp.int32) + stride
        return carry

    lax.fori_loop(0, _NGROUP, expand, None)

    copies = []
    for j in range(_NCHUNK):
        sl = pl.ds(j * _CHUNK, _CHUNK)
        copies.append(pltpu.make_async_copy(cust_hbm.at[xe.at[sl]], xrows.at[sl], sem))
        copies.append(pltpu.make_async_copy(art_hbm.at[ye.at[sl]], yrows.at[sl], sem))
    for w in range(0, len(copies), _WAVE):
        wave = copies[w:w + _WAVE]
        for c in wave:
            c.start()
        for c in wave:
            c.wait()

    def group(g, carry):
        fb = g * 256 + iota * DIM
        acc = jnp.zeros((16,), jnp.float32)
        for d in range(DIM):
            flat = fb + lax.bitwise_and(iota + d, 15)
            xa = plsc.load_gather(xrows, [flat])
            ya = plsc.load_gather(yrows, [flat])
            acc = acc + xa * ya
        out_v[pl.ds(g * 16, 16)] = acc
        return carry

    lax.fori_loop(0, _NGROUP, group, None)

    pltpu.sync_copy(out_v, out_hbm.at[pl.ds(base, _ROWS_PER_W)])


def kernel(x, y, article_table, customer_table):
    x = x.astype(jnp.int32)
    y = y.astype(jnp.int32)
    art_flat = article_table.T.reshape(-1)
    cust_flat = customer_table.T.reshape(-1)
    mesh = plsc.VectorSubcoreMesh(
        core_axis_name="c", subcore_axis_name="s",
        num_cores=_NC, num_subcores=_NS)
    run = pl.kernel(
        _tt_body,
        out_type=jax.ShapeDtypeStruct((BATCH,), jnp.float32),
        mesh=mesh,
        scratch_types=[
            pltpu.VMEM((_ROWS_PER_W,), jnp.int32),
            pltpu.VMEM((_ROWS_PER_W,), jnp.int32),
            pltpu.VMEM((_EL_PER_W,), jnp.int32),
            pltpu.VMEM((_EL_PER_W,), jnp.int32),
            pltpu.VMEM((_EL_PER_W,), jnp.float32),
            pltpu.VMEM((_EL_PER_W,), jnp.float32),
            pltpu.VMEM((_ROWS_PER_W,), jnp.float32),
            pltpu.SemaphoreType.DMA,
        ],
        compiler_params=pltpu.CompilerParams(
            needs_layout_passes=False, use_tc_tiling_on_sc=False),
    )
    return run(x, y, art_flat, cust_flat)


# speedup vs baseline: 3.1922x; 3.1922x over previous
import jax
import jax.numpy as jnp
from jax import lax
from jax.experimental import pallas as pl
from jax.experimental.pallas import tpu as pltpu
from jax.experimental.pallas import tpu_sc as plsc

BATCH = 16384
DIM = 16

_NC = 2
_NS = 16
_NW = _NC * _NS
_ROWS_PER_W = BATCH // _NW
_CHUNK = 128
_NCHUNK = _ROWS_PER_W // _CHUNK
_NGROUP = _ROWS_PER_W // 16


def _tt_body(x_hbm, y_hbm, art_hbm, cust_hbm, out_hbm,
             xidx, yidx, xrows, yrows, out_v, sem):
    wid = lax.axis_index("s") * _NC + lax.axis_index("c")
    base = wid * _ROWS_PER_W

    idx_copies = []
    for j in range(_NCHUNK):
        src = pl.ds(base + j * _CHUNK, _CHUNK)
        idx_copies.append(pltpu.make_async_copy(x_hbm.at[src], xidx.at[j], sem))
        idx_copies.append(pltpu.make_async_copy(y_hbm.at[src], yidx.at[j], sem))
    for c in idx_copies:
        c.start()
    for c in idx_copies:
        c.wait()

    row_copies = []
    for j in range(_NCHUNK):
        dst = pl.ds(j * _CHUNK, _CHUNK)
        row_copies.append(
            pltpu.make_async_copy(cust_hbm.at[xidx.at[j]], xrows.at[dst], sem))
        row_copies.append(
            pltpu.make_async_copy(art_hbm.at[yidx.at[j]], yrows.at[dst], sem))
    for c in row_copies:
        c.start()
    for c in row_copies:
        c.wait()

    iota = lax.iota(jnp.int32, 16)

    def group(g, carry):
        rows = g * 16 + iota
        acc = jnp.zeros((16,), jnp.float32)
        for d in range(DIM):
            cols = lax.bitwise_and(iota + d, 15)
            xa = plsc.load_gather(xrows, [rows, cols])
            ya = plsc.load_gather(yrows, [rows, cols])
            acc = acc + xa * ya
        out_v[pl.ds(g * 16, 16)] = acc
        return carry

    lax.fori_loop(0, _NGROUP, group, None)

    pltpu.sync_copy(out_v, out_hbm.at[pl.ds(base, _ROWS_PER_W)])


def kernel(x, y, article_table, customer_table):
    x = x.astype(jnp.int32)
    y = y.astype(jnp.int32)
    mesh = plsc.VectorSubcoreMesh(
        core_axis_name="c", subcore_axis_name="s",
        num_cores=_NC, num_subcores=_NS)
    run = pl.kernel(
        _tt_body,
        out_type=jax.ShapeDtypeStruct((BATCH,), jnp.float32),
        mesh=mesh,
        scratch_types=[
            pltpu.VMEM((_NCHUNK, _CHUNK), jnp.int32),
            pltpu.VMEM((_NCHUNK, _CHUNK), jnp.int32),
            pltpu.VMEM((_ROWS_PER_W, DIM), jnp.float32),
            pltpu.VMEM((_ROWS_PER_W, DIM), jnp.float32),
            pltpu.VMEM((_ROWS_PER_W,), jnp.float32),
            pltpu.SemaphoreType.DMA,
        ],
        compiler_params=pltpu.CompilerParams(
            needs_layout_passes=False, use_tc_tiling_on_sc=False),
    )
    return run(x, y, article_table, customer_table)
